# Initial kernel scaffold; baseline (speedup 1.0000x reference)
import jax
import jax.numpy as jnp
from jax.experimental import pallas as pl


def kernel(x, edge_index, batch, weight, w_ih, w_hh, b_ih, b_hh, fc1_w, fc1_b, bn_g, bn_b, fc2_w, fc2_b):
    raise NotImplementedError("write your pallas kernel here")


# speedup vs baseline: 9.7528x; 9.7528x over previous
import functools

import jax
import jax.numpy as jnp
from jax import lax
from jax.experimental import pallas as pl
from jax.experimental.pallas import tpu as pltpu
from jax.experimental.pallas import tpu_sc as plsc

N = 10000
H = 128
G = 64

_NC = 2
_NS = 16
_CHUNK = 128
_KMAX = -(-(320000 // _CHUNK // _NC) // _NS)

_DOT = dict(preferred_element_type=jnp.float32,
            precision=jax.lax.Precision.HIGHEST)


def _seg_sum_body(msg_hbm, src_hbm, dst_hbm, out_hbm, sidx, didx,
                  rows0, rows1, acc, sem0, sem1, semi0, semi1):
    c = lax.axis_index("c")
    s = lax.axis_index("s")
    E = src_hbm.shape[0]
    n_chunks = E // _CHUNK
    per_core = n_chunks // _NC
    kmax = (per_core + _NS - 1) // _NS
    slab = (N // 8 // _NS) * 8
    chunks = []
    o = 0
    while o < slab:
        sz = min(_CHUNK, slab - o)
        chunks.append((o, sz))
        o += sz

    @pl.loop(0, _CHUNK)
    def _(r):
        @pl.loop(0, H // 16)
        def _(j):
            rows0[r, pl.ds(j * 16, 16)] = jnp.zeros((16,), jnp.float32)

    row0 = pl.multiple_of(s * slab, 8)
    for o, sz in chunks:
        pltpu.sync_copy(rows0.at[pl.ds(0, sz)],
                        acc.at[pl.ds(row0 + o, sz)])

    @pl.when(s < (N - slab * _NS) // 8)
    def _():
        r0 = pl.multiple_of(slab * _NS + s * 8, 8)
        pltpu.sync_copy(rows0.at[pl.ds(0, 8)], acc.at[pl.ds(r0, 8)])

    base_k = per_core // _NS
    extra = per_core - base_k * _NS
    cs = c * per_core + s * base_k + jnp.minimum(s, extra)
    kn = base_k + jnp.where(s < extra, 1, 0)
    edge0 = cs * _CHUNK

    @pl.when(s < extra)
    def _():
        pltpu.sync_copy(src_hbm.at[pl.ds(edge0, (base_k + 1) * _CHUNK)], sidx)

    @pl.when(s >= extra)
    def _():
        pltpu.sync_copy(src_hbm.at[pl.ds(edge0, base_k * _CHUNK)],
                        sidx.at[pl.ds(0, base_k * _CHUNK)])

    plsc.subcore_barrier()

    bufs = ((rows0, sem0, semi0), (rows1, sem1, semi1))

    def _didx_copy(k, b, semi_b):
        return pltpu.make_async_copy(
            dst_hbm.at[pl.ds(edge0 + k * _CHUNK, _CHUNK)],
            didx.at[b], semi_b)

    def _gather_copy(k, rows_b, sem_b):
        return pltpu.make_async_copy(
            msg_hbm.at[sidx.at[pl.ds(k * _CHUNK, _CHUNK)]],
            rows_b, sem_b)

    for b in (0, 1):
        rows_b, sem_b, semi_b = bufs[b]

        @pl.when(b < kn)
        def _(b=b, rows_b=rows_b, sem_b=sem_b, semi_b=semi_b):
            _didx_copy(b, b, semi_b).start()
            _gather_copy(b, rows_b, sem_b).start()

    @pl.loop(0, (kmax + 1) // 2)
    def _(k2):
        for b in (0, 1):
            rows_b, sem_b, semi_b = bufs[b]
            k = k2 * 2 + b

            @pl.when(k < kn)
            def _(k=k, b=b, rows_b=rows_b, sem_b=sem_b, semi_b=semi_b):
                _didx_copy(k, b, semi_b).wait()
                _gather_copy(k, rows_b, sem_b).wait()
                pltpu.sync_copy(rows_b, acc.at[didx.at[b]], add=True)

                @pl.when(k + 2 < kn)
                def _():
                    _didx_copy(k + 2, b, semi_b).start()
                    _gather_copy(k + 2, rows_b, sem_b).start()

    plsc.subcore_barrier()

    for o, sz in chunks:
        r1 = pl.multiple_of(row0 + o, 8)
        pltpu.sync_copy(acc.at[pl.ds(r1, sz)],
                        out_hbm.at[pl.ds(pl.multiple_of(c * N + r1, 8), sz)])

    @pl.when(s < (N - slab * _NS) // 8)
    def _():
        r0 = pl.multiple_of(slab * _NS + s * 8, 8)
        pltpu.sync_copy(acc.at[pl.ds(r0, 8)],
                        out_hbm.at[pl.ds(pl.multiple_of(c * N + r0, 8), 8)])


def _seg_sum(msg, src, dst):
    mesh = plsc.VectorSubcoreMesh(core_axis_name="c", subcore_axis_name="s")
    f = pl.kernel(
        _seg_sum_body,
        out_type=jax.ShapeDtypeStruct((_NC * N, H), jnp.float32),
        mesh=mesh,
        scratch_types=[
            pltpu.VMEM((_KMAX * _CHUNK,), jnp.int32),
            pltpu.VMEM((2, _CHUNK), jnp.int32),
            pltpu.VMEM((_CHUNK, H), jnp.float32),
            pltpu.VMEM((_CHUNK, H), jnp.float32),
            pltpu.VMEM_SHARED((N, H), jnp.float32),
            pltpu.SemaphoreType.DMA,
            pltpu.SemaphoreType.DMA,
            pltpu.SemaphoreType.DMA,
            pltpu.SemaphoreType.DMA,
        ],
    )
    return f(msg, src, dst)


_RB = 2000


def _gru_body(p0_ref, p1_ref, h_ref, w_ref, wih_ref, whh_ref, bih_ref,
              bhh_ref, o_ref):
    agg = p0_ref[...] + p1_ref[...]
    m = lax.dot_general(agg, w_ref[...], (((1,), (0,)), ((), ())), **_DOT)
    h = h_ref[...]
    gi = lax.dot_general(m, wih_ref[...], (((1,), (1,)), ((), ())), **_DOT)
    gi = gi + bih_ref[...][None, :]
    gh = lax.dot_general(h, whh_ref[...], (((1,), (1,)), ((), ())), **_DOT)
    gh = gh + bhh_ref[...][None, :]
    r = jax.nn.sigmoid(gi[:, 0:H] + gh[:, 0:H])
    z = jax.nn.sigmoid(gi[:, H:2 * H] + gh[:, H:2 * H])
    n = jnp.tanh(gi[:, 2 * H:3 * H] + r * gh[:, 2 * H:3 * H])
    o_ref[...] = (1.0 - z) * n + z * h


def _gru_call(parts, h, w, w_ih, w_hh, b_ih, b_hh):
    nb = N // _RB
    return pl.pallas_call(
        _gru_body,
        grid=(nb,),
        in_specs=[pl.BlockSpec((_RB, H), lambda i: (i, 0)),
                  pl.BlockSpec((_RB, H), lambda i, nb=nb: (i + nb, 0)),
                  pl.BlockSpec((_RB, H), lambda i: (i, 0)),
                  pl.BlockSpec((H, H), lambda i: (0, 0)),
                  pl.BlockSpec((3 * H, H), lambda i: (0, 0)),
                  pl.BlockSpec((3 * H, H), lambda i: (0, 0)),
                  pl.BlockSpec((3 * H,), lambda i: (0,)),
                  pl.BlockSpec((3 * H,), lambda i: (0,))],
        out_specs=pl.BlockSpec((_RB, H), lambda i: (i, 0)),
        out_shape=jax.ShapeDtypeStruct((N, H), jnp.float32),
    )(parts, parts, h, w, w_ih, w_hh, b_ih, b_hh)


def _fc1_body(h_ref, w_ref, b_ref, y_ref, st_ref):
    i = pl.program_id(0)
    y = lax.dot_general(h_ref[...], w_ref[...], (((1,), (1,)), ((), ())),
                        **_DOT) + b_ref[...][None, :]
    y_ref[...] = y
    su = jnp.sum(y, axis=0)
    sq = jnp.sum(y * y, axis=0)
    st = jnp.stack([su, sq], axis=0)

    @pl.when(i == 0)
    def _():
        st_ref[...] = st

    @pl.when(i != 0)
    def _():
        st_ref[...] += st


def _fc1_call(h, fc1_w, fc1_b):
    return pl.pallas_call(
        _fc1_body,
        grid=(N // _RB,),
        in_specs=[pl.BlockSpec((_RB, H), lambda i: (i, 0)),
                  pl.BlockSpec((H, H), lambda i: (0, 0)),
                  pl.BlockSpec((H,), lambda i: (0,))],
        out_specs=[pl.BlockSpec((_RB, H), lambda i: (i, 0)),
                   pl.BlockSpec((2, H), lambda i: (0, 0))],
        out_shape=[jax.ShapeDtypeStruct((N, H), jnp.float32),
                   jax.ShapeDtypeStruct((2, H), jnp.float32)],
    )(h, fc1_w, fc1_b)


def _pool_body(y_ref, st_ref, bng_ref, bnb_ref, batch_ref, gs_ref, gc_ref):
    i = pl.program_id(0)
    mean = st_ref[0, :] / N
    var = st_ref[1, :] / N - mean * mean
    scale = bng_ref[...] * lax.rsqrt(var + 1e-5)
    y = (y_ref[...] - mean[None, :]) * scale[None, :] + bnb_ref[...][None, :]
    y = jnp.maximum(y, 0.0)
    b = batch_ref[0, 0, :]
    onehot = (b[:, None] == lax.broadcasted_iota(jnp.int32, (_RB, G), 1))
    onehot = onehot.astype(jnp.float32)
    gs = lax.dot_general(onehot, y, (((0,), (0,)), ((), ())), **_DOT)
    gc = jnp.sum(onehot, axis=0)[None, :]

    @pl.when(i == 0)
    def _():
        gs_ref[...] = gs
        gc_ref[...] = gc

    @pl.when(i != 0)
    def _():
        gs_ref[...] += gs
        gc_ref[...] += gc


def _pool_call(y, stats, bn_g, bn_b, batch):
    return pl.pallas_call(
        _pool_body,
        grid=(N // _RB,),
        in_specs=[pl.BlockSpec((_RB, H), lambda i: (i, 0)),
                  pl.BlockSpec((2, H), lambda i: (0, 0)),
                  pl.BlockSpec((H,), lambda i: (0,)),
                  pl.BlockSpec((H,), lambda i: (0,)),
                  pl.BlockSpec((1, 1, _RB), lambda i: (i, 0, 0))],
        out_specs=[pl.BlockSpec((G, H), lambda i: (0, 0)),
                   pl.BlockSpec((1, G), lambda i: (0, 0))],
        out_shape=[jax.ShapeDtypeStruct((G, H), jnp.float32),
                   jax.ShapeDtypeStruct((1, G), jnp.float32)],
    )(y, stats, bn_g, bn_b, batch.reshape(N // _RB, 1, _RB))


def _head_body(gs_ref, gc_ref, w_ref, b_ref, o_ref):
    cnt = jnp.maximum(gc_ref[0, :], 1.0)
    gm = gs_ref[...] / cnt[:, None]
    logits = lax.dot_general(gm, w_ref[...], (((1,), (1,)), ((), ())),
                             **_DOT) + b_ref[...][None, :]
    mx = jnp.max(logits, axis=-1, keepdims=True)
    sh = logits - mx
    lse = jnp.log(jnp.sum(jnp.exp(sh), axis=-1, keepdims=True))
    o_ref[...] = sh - lse


def _head_call(gs, gc, fc2_w, fc2_b):
    C = fc2_w.shape[0]
    return pl.pallas_call(
        _head_body,
        out_shape=jax.ShapeDtypeStruct((G, C), jnp.float32),
    )(gs, gc, fc2_w, fc2_b)


def kernel(x, edge_index, batch, weight, w_ih, w_hh, b_ih, b_hh,
           fc1_w, fc1_b, bn_g, bn_b, fc2_w, fc2_b):
    src = edge_index[0]
    dst = edge_index[1]
    h = x
    for l in range(weight.shape[0]):
        parts = _seg_sum(h, src, dst)
        h = _gru_call(parts, h, weight[l], w_ih, w_hh, b_ih, b_hh)
    y, stats = _fc1_call(h, fc1_w, fc1_b)
    gs, gc = _pool_call(y, stats, bn_g, bn_b, batch)
    return _head_call(gs, gc, fc2_w, fc2_b)


# speedup vs baseline: 12.2178x; 1.2528x over previous
import functools

import jax
import jax.numpy as jnp
from jax import lax
from jax.experimental import pallas as pl
from jax.experimental.pallas import tpu as pltpu
from jax.experimental.pallas import tpu_sc as plsc

N = 10000
H = 128
G = 64

_NC = 2
_NS = 16
_CHUNK = 128
_KMAX = -(-(320000 // _CHUNK // _NC) // _NS)

_DOT = dict(preferred_element_type=jnp.float32,
            precision=jax.lax.Precision.DEFAULT)


def _seg_sum_body(msg_hbm, src_hbm, dst_hbm, out_hbm, sidx, didx,
                  rows0, rows1, acc, sem0, sem1, semi0, semi1):
    c = lax.axis_index("c")
    s = lax.axis_index("s")
    E = src_hbm.shape[0]
    n_chunks = E // _CHUNK
    per_core = n_chunks // _NC
    kmax = (per_core + _NS - 1) // _NS
    slab = (N // 8 // _NS) * 8
    chunks = []
    o = 0
    while o < slab:
        sz = min(_CHUNK, slab - o)
        chunks.append((o, sz))
        o += sz

    @pl.loop(0, _CHUNK)
    def _(r):
        @pl.loop(0, H // 16)
        def _(j):
            rows0[r, pl.ds(j * 16, 16)] = jnp.zeros((16,), jnp.float32)

    row0 = pl.multiple_of(s * slab, 8)
    for o, sz in chunks:
        pltpu.sync_copy(rows0.at[pl.ds(0, sz)],
                        acc.at[pl.ds(row0 + o, sz)])

    @pl.when(s < (N - slab * _NS) // 8)
    def _():
        r0 = pl.multiple_of(slab * _NS + s * 8, 8)
        pltpu.sync_copy(rows0.at[pl.ds(0, 8)], acc.at[pl.ds(r0, 8)])

    base_k = per_core // _NS
    extra = per_core - base_k * _NS
    cs = c * per_core + s * base_k + jnp.minimum(s, extra)
    kn = base_k + jnp.where(s < extra, 1, 0)
    edge0 = cs * _CHUNK

    @pl.when(s < extra)
    def _():
        pltpu.sync_copy(src_hbm.at[pl.ds(edge0, (base_k + 1) * _CHUNK)], sidx)

    @pl.when(s >= extra)
    def _():
        pltpu.sync_copy(src_hbm.at[pl.ds(edge0, base_k * _CHUNK)],
                        sidx.at[pl.ds(0, base_k * _CHUNK)])

    plsc.subcore_barrier()

    bufs = ((rows0, sem0, semi0), (rows1, sem1, semi1))

    def _didx_copy(k, b, semi_b):
        return pltpu.make_async_copy(
            dst_hbm.at[pl.ds(edge0 + k * _CHUNK, _CHUNK)],
            didx.at[b], semi_b)

    def _gather_copy(k, rows_b, sem_b):
        return pltpu.make_async_copy(
            msg_hbm.at[sidx.at[pl.ds(k * _CHUNK, _CHUNK)]],
            rows_b, sem_b)

    for b in (0, 1):
        rows_b, sem_b, semi_b = bufs[b]

        @pl.when(b < kn)
        def _(b=b, rows_b=rows_b, sem_b=sem_b, semi_b=semi_b):
            _didx_copy(b, b, semi_b).start()
            _gather_copy(b, rows_b, sem_b).start()

    @pl.loop(0, (kmax + 1) // 2)
    def _(k2):
        for b in (0, 1):
            rows_b, sem_b, semi_b = bufs[b]
            k = k2 * 2 + b

            @pl.when(k < kn)
            def _(k=k, b=b, rows_b=rows_b, sem_b=sem_b, semi_b=semi_b):
                _didx_copy(k, b, semi_b).wait()
                _gather_copy(k, rows_b, sem_b).wait()
                pltpu.sync_copy(rows_b, acc.at[didx.at[b]], add=True)

                @pl.when(k + 2 < kn)
                def _():
                    _didx_copy(k + 2, b, semi_b).start()
                    _gather_copy(k + 2, rows_b, sem_b).start()

    plsc.subcore_barrier()

    for o, sz in chunks:
        r1 = pl.multiple_of(row0 + o, 8)
        pltpu.sync_copy(acc.at[pl.ds(r1, sz)],
                        out_hbm.at[pl.ds(pl.multiple_of(c * N + r1, 8), sz)])

    @pl.when(s < (N - slab * _NS) // 8)
    def _():
        r0 = pl.multiple_of(slab * _NS + s * 8, 8)
        pltpu.sync_copy(acc.at[pl.ds(r0, 8)],
                        out_hbm.at[pl.ds(pl.multiple_of(c * N + r0, 8), 8)])


def _seg_sum(msg, src, dst):
    mesh = plsc.VectorSubcoreMesh(core_axis_name="c", subcore_axis_name="s")
    f = pl.kernel(
        _seg_sum_body,
        out_type=jax.ShapeDtypeStruct((_NC * N, H), jnp.float32),
        mesh=mesh,
        scratch_types=[
            pltpu.VMEM((_KMAX * _CHUNK,), jnp.int32),
            pltpu.VMEM((2, _CHUNK), jnp.int32),
            pltpu.VMEM((_CHUNK, H), jnp.float32),
            pltpu.VMEM((_CHUNK, H), jnp.float32),
            pltpu.VMEM_SHARED((N, H), jnp.float32),
            pltpu.SemaphoreType.DMA,
            pltpu.SemaphoreType.DMA,
            pltpu.SemaphoreType.DMA,
            pltpu.SemaphoreType.DMA,
        ],
    )
    return f(msg, src, dst)


_RB = 2000


def _gru_body(p0_ref, p1_ref, h_ref, w_ref, wih_ref, whh_ref, bih_ref,
              bhh_ref, o_ref):
    agg = p0_ref[...] + p1_ref[...]
    m = lax.dot_general(agg, w_ref[...], (((1,), (0,)), ((), ())), **_DOT)
    h = h_ref[...]
    gi = lax.dot_general(m, wih_ref[...], (((1,), (1,)), ((), ())), **_DOT)
    gi = gi + bih_ref[...][None, :]
    gh = lax.dot_general(h, whh_ref[...], (((1,), (1,)), ((), ())), **_DOT)
    gh = gh + bhh_ref[...][None, :]
    r = jax.nn.sigmoid(gi[:, 0:H] + gh[:, 0:H])
    z = jax.nn.sigmoid(gi[:, H:2 * H] + gh[:, H:2 * H])
    n = jnp.tanh(gi[:, 2 * H:3 * H] + r * gh[:, 2 * H:3 * H])
    o_ref[...] = (1.0 - z) * n + z * h


def _gru_call(parts, h, w, w_ih, w_hh, b_ih, b_hh):
    nb = N // _RB
    return pl.pallas_call(
        _gru_body,
        grid=(nb,),
        in_specs=[pl.BlockSpec((_RB, H), lambda i: (i, 0)),
                  pl.BlockSpec((_RB, H), lambda i, nb=nb: (i + nb, 0)),
                  pl.BlockSpec((_RB, H), lambda i: (i, 0)),
                  pl.BlockSpec((H, H), lambda i: (0, 0)),
                  pl.BlockSpec((3 * H, H), lambda i: (0, 0)),
                  pl.BlockSpec((3 * H, H), lambda i: (0, 0)),
                  pl.BlockSpec((3 * H,), lambda i: (0,)),
                  pl.BlockSpec((3 * H,), lambda i: (0,))],
        out_specs=pl.BlockSpec((_RB, H), lambda i: (i, 0)),
        out_shape=jax.ShapeDtypeStruct((N, H), jnp.float32),
    )(parts, parts, h, w, w_ih, w_hh, b_ih, b_hh)


def _tail_body(h_ref, w1_ref, b1_ref, bng_ref, bnb_ref, batch_ref, w2_ref,
               b2_ref, o_ref):
    h = h_ref[...]
    y = lax.dot_general(h, w1_ref[...], (((1,), (1,)), ((), ())),
                        **_DOT) + b1_ref[...][None, :]
    mean = jnp.sum(y, axis=0) / N
    var = jnp.sum(y * y, axis=0) / N - mean * mean
    scale = bng_ref[...] * lax.rsqrt(var + 1e-5)
    y = (y - mean[None, :]) * scale[None, :] + bnb_ref[...][None, :]
    y = jnp.maximum(y, 0.0)
    b = batch_ref[0, :]
    onehot = (b[:, None] == lax.broadcasted_iota(jnp.int32, (N, G), 1))
    onehot = onehot.astype(jnp.float32)
    gs = lax.dot_general(onehot, y, (((0,), (0,)), ((), ())), **_DOT)
    gc = jnp.sum(onehot, axis=0)
    gm = gs / jnp.maximum(gc, 1.0)[:, None]
    logits = lax.dot_general(gm, w2_ref[...], (((1,), (1,)), ((), ())),
                             **_DOT) + b2_ref[...][None, :]
    mx = jnp.max(logits, axis=-1, keepdims=True)
    sh = logits - mx
    lse = jnp.log(jnp.sum(jnp.exp(sh), axis=-1, keepdims=True))
    o_ref[...] = sh - lse


def _tail_call(h, fc1_w, fc1_b, bn_g, bn_b, batch, fc2_w, fc2_b):
    C = fc2_w.shape[0]
    return pl.pallas_call(
        _tail_body,
        out_shape=jax.ShapeDtypeStruct((G, C), jnp.float32),
    )(h, fc1_w, fc1_b, bn_g, bn_b, batch.reshape(1, N), fc2_w, fc2_b)


def kernel(x, edge_index, batch, weight, w_ih, w_hh, b_ih, b_hh,
           fc1_w, fc1_b, bn_g, bn_b, fc2_w, fc2_b):
    src = edge_index[0]
    dst = edge_index[1]
    h = x
    for l in range(weight.shape[0]):
        parts = _seg_sum(h, src, dst)
        h = _gru_call(parts, h, weight[l], w_ih, w_hh, b_ih, b_hh)
    return _tail_call(h, fc1_w, fc1_b, bn_g, bn_b, batch, fc2_w, fc2_b)


# speedup vs baseline: 12.3429x; 1.0102x over previous
import functools

import jax
import jax.numpy as jnp
from jax import lax
from jax.experimental import pallas as pl
from jax.experimental.pallas import tpu as pltpu
from jax.experimental.pallas import tpu_sc as plsc

N = 10000
H = 128
G = 64

_NC = 2
_NS = 16
_CHUNK = 128
_KMAX = -(-(320000 // _CHUNK // _NC) // _NS)

_DOT = dict(preferred_element_type=jnp.float32,
            precision=jax.lax.Precision.DEFAULT)


_NBUF = 3


def _seg_sum_body(msg_hbm, src_hbm, dst_hbm, out_hbm, sidx, didx,
                  rows0, rows1, rows2, acc,
                  sg0, sg1, sg2, si0, si1, si2):
    c = lax.axis_index("c")
    s = lax.axis_index("s")
    E = src_hbm.shape[0]
    n_chunks = E // _CHUNK
    per_core = n_chunks // _NC
    kmax = (per_core + _NS - 1) // _NS
    slab = (N // 8 // _NS) * 8
    chunks = []
    o = 0
    while o < slab:
        sz = min(_CHUNK, slab - o)
        chunks.append((o, sz))
        o += sz

    @pl.loop(0, _CHUNK)
    def _(r):
        @pl.loop(0, H // 16)
        def _(j):
            rows0[r, pl.ds(j * 16, 16)] = jnp.zeros((16,), jnp.float32)

    row0 = pl.multiple_of(s * slab, 8)
    for o, sz in chunks:
        pltpu.sync_copy(rows0.at[pl.ds(0, sz)],
                        acc.at[pl.ds(row0 + o, sz)])

    @pl.when(s == 0)
    def _():
        r0 = pl.multiple_of(slab * _NS, 8)
        pltpu.sync_copy(rows0.at[pl.ds(0, N - slab * _NS)],
                        acc.at[pl.ds(r0, N - slab * _NS)])

    base_k = per_core // _NS
    extra = per_core - base_k * _NS
    cs = c * per_core + s * base_k + jnp.minimum(s, extra)
    kn = base_k + jnp.where(s < extra, 1, 0)
    edge0 = cs * _CHUNK

    plsc.subcore_barrier()

    bufs = ((rows0, sg0, si0), (rows1, sg1, si1), (rows2, sg2, si2))

    def _idx_copies(k, b, si_b):
        return (pltpu.make_async_copy(
                    src_hbm.at[pl.ds(edge0 + k * _CHUNK, _CHUNK)],
                    sidx.at[b], si_b),
                pltpu.make_async_copy(
                    dst_hbm.at[pl.ds(edge0 + k * _CHUNK, _CHUNK)],
                    didx.at[b], si_b))

    def _gather_copy(k, b, rows_b, sg_b):
        return pltpu.make_async_copy(
            msg_hbm.at[sidx.at[b]], rows_b, sg_b)

    for b in range(_NBUF):
        rows_b, sg_b, si_b = bufs[b]

        @pl.when(b < kn)
        def _(b=b, si_b=si_b):
            for cp in _idx_copies(b, b, si_b):
                cp.start()

    for b in range(2):
        rows_b, sg_b, si_b = bufs[b]

        @pl.when(b < kn)
        def _(b=b, rows_b=rows_b, sg_b=sg_b, si_b=si_b):
            for cp in _idx_copies(b, b, si_b):
                cp.wait()
            _gather_copy(b, b, rows_b, sg_b).start()

    @pl.loop(0, (kmax + _NBUF - 1) // _NBUF)
    def _(kq):
        for b in range(_NBUF):
            rows_b, sg_b, si_b = bufs[b]
            b2 = (b + 2) % _NBUF
            rows_b2, sg_b2, si_b2 = bufs[b2]
            k = kq * _NBUF + b

            @pl.when(k < kn)
            def _(k=k, b=b, rows_b=rows_b, sg_b=sg_b, si_b=si_b,
                  b2=b2, rows_b2=rows_b2, sg_b2=sg_b2, si_b2=si_b2):
                _gather_copy(k, b, rows_b, sg_b).wait()
                pltpu.sync_copy(rows_b, acc.at[didx.at[b]], add=True)

                @pl.when(k + _NBUF < kn)
                def _():
                    for cp in _idx_copies(k + _NBUF, b, si_b):
                        cp.start()

                @pl.when(k + 2 < kn)
                def _():
                    for cp in _idx_copies(k + 2, b2, si_b2):
                        cp.wait()
                    _gather_copy(k + 2, b2, rows_b2, sg_b2).start()

    plsc.subcore_barrier()

    for o, sz in chunks:
        r1 = pl.multiple_of(row0 + o, 8)
        pltpu.sync_copy(acc.at[pl.ds(r1, sz)],
                        out_hbm.at[pl.ds(pl.multiple_of(c * N + r1, 8), sz)])

    @pl.when(s == 0)
    def _():
        r0 = pl.multiple_of(slab * _NS, 8)
        pltpu.sync_copy(
            acc.at[pl.ds(r0, N - slab * _NS)],
            out_hbm.at[pl.ds(pl.multiple_of(c * N + r0, 8), N - slab * _NS)])


def _seg_sum(msg, src, dst):
    mesh = plsc.VectorSubcoreMesh(core_axis_name="c", subcore_axis_name="s")
    f = pl.kernel(
        _seg_sum_body,
        out_type=jax.ShapeDtypeStruct((_NC * N, H), jnp.float32),
        mesh=mesh,
        scratch_types=[
            pltpu.VMEM((_NBUF, _CHUNK), jnp.int32),
            pltpu.VMEM((_NBUF, _CHUNK), jnp.int32),
            pltpu.VMEM((_CHUNK, H), jnp.float32),
            pltpu.VMEM((_CHUNK, H), jnp.float32),
            pltpu.VMEM((_CHUNK, H), jnp.float32),
            pltpu.VMEM_SHARED((N, H), jnp.float32),
            pltpu.SemaphoreType.DMA,
            pltpu.SemaphoreType.DMA,
            pltpu.SemaphoreType.DMA,
            pltpu.SemaphoreType.DMA,
            pltpu.SemaphoreType.DMA,
            pltpu.SemaphoreType.DMA,
        ],
    )
    return f(msg, src, dst)


_RB = 2000


def _gru_body(p0_ref, p1_ref, h_ref, w_ref, wih_ref, whh_ref, bih_ref,
              bhh_ref, o_ref):
    agg = p0_ref[...] + p1_ref[...]
    m = lax.dot_general(agg, w_ref[...], (((1,), (0,)), ((), ())), **_DOT)
    h = h_ref[...]
    gi = lax.dot_general(m, wih_ref[...], (((1,), (1,)), ((), ())), **_DOT)
    gi = gi + bih_ref[...][None, :]
    gh = lax.dot_general(h, whh_ref[...], (((1,), (1,)), ((), ())), **_DOT)
    gh = gh + bhh_ref[...][None, :]
    r = jax.nn.sigmoid(gi[:, 0:H] + gh[:, 0:H])
    z = jax.nn.sigmoid(gi[:, H:2 * H] + gh[:, H:2 * H])
    n = jnp.tanh(gi[:, 2 * H:3 * H] + r * gh[:, 2 * H:3 * H])
    o_ref[...] = (1.0 - z) * n + z * h


def _gru_call(parts, h, w, w_ih, w_hh, b_ih, b_hh):
    nb = N // _RB
    return pl.pallas_call(
        _gru_body,
        grid=(nb,),
        in_specs=[pl.BlockSpec((_RB, H), lambda i: (i, 0)),
                  pl.BlockSpec((_RB, H), lambda i, nb=nb: (i + nb, 0)),
                  pl.BlockSpec((_RB, H), lambda i: (i, 0)),
                  pl.BlockSpec((H, H), lambda i: (0, 0)),
                  pl.BlockSpec((3 * H, H), lambda i: (0, 0)),
                  pl.BlockSpec((3 * H, H), lambda i: (0, 0)),
                  pl.BlockSpec((3 * H,), lambda i: (0,)),
                  pl.BlockSpec((3 * H,), lambda i: (0,))],
        out_specs=pl.BlockSpec((_RB, H), lambda i: (i, 0)),
        out_shape=jax.ShapeDtypeStruct((N, H), jnp.float32),
    )(parts, parts, h, w, w_ih, w_hh, b_ih, b_hh)


def _tail_body(h_ref, w1_ref, b1_ref, bng_ref, bnb_ref, batch_ref, w2_ref,
               b2_ref, o_ref):
    h = h_ref[...]
    y = lax.dot_general(h, w1_ref[...], (((1,), (1,)), ((), ())),
                        **_DOT) + b1_ref[...][None, :]
    mean = jnp.sum(y, axis=0) / N
    var = jnp.sum(y * y, axis=0) / N - mean * mean
    scale = bng_ref[...] * lax.rsqrt(var + 1e-5)
    y = (y - mean[None, :]) * scale[None, :] + bnb_ref[...][None, :]
    y = jnp.maximum(y, 0.0)
    b = batch_ref[0, :]
    onehot = (b[:, None] == lax.broadcasted_iota(jnp.int32, (N, G), 1))
    onehot = onehot.astype(jnp.float32)
    gs = lax.dot_general(onehot, y, (((0,), (0,)), ((), ())), **_DOT)
    gc = jnp.sum(onehot, axis=0)
    gm = gs / jnp.maximum(gc, 1.0)[:, None]
    logits = lax.dot_general(gm, w2_ref[...], (((1,), (1,)), ((), ())),
                             **_DOT) + b2_ref[...][None, :]
    mx = jnp.max(logits, axis=-1, keepdims=True)
    sh = logits - mx
    lse = jnp.log(jnp.sum(jnp.exp(sh), axis=-1, keepdims=True))
    o_ref[...] = sh - lse


def _tail_call(h, fc1_w, fc1_b, bn_g, bn_b, batch, fc2_w, fc2_b):
    C = fc2_w.shape[0]
    return pl.pallas_call(
        _tail_body,
        out_shape=jax.ShapeDtypeStruct((G, C), jnp.float32),
    )(h, fc1_w, fc1_b, bn_g, bn_b, batch.reshape(1, N), fc2_w, fc2_b)


def kernel(x, edge_index, batch, weight, w_ih, w_hh, b_ih, b_hh,
           fc1_w, fc1_b, bn_g, bn_b, fc2_w, fc2_b):
    src = edge_index[0]
    dst = edge_index[1]
    h = x
    for l in range(weight.shape[0]):
        parts = _seg_sum(h, src, dst)
        h = _gru_call(parts, h, weight[l], w_ih, w_hh, b_ih, b_hh)
    return _tail_call(h, fc1_w, fc1_b, bn_g, bn_b, batch, fc2_w, fc2_b)


# speedup vs baseline: 12.4357x; 1.0075x over previous
import functools

import jax
import jax.numpy as jnp
from jax import lax
from jax.experimental import pallas as pl
from jax.experimental.pallas import tpu as pltpu
from jax.experimental.pallas import tpu_sc as plsc

N = 10000
H = 128
G = 64

_NC = 2
_NS = 16
_CHUNK = 128
_KMAX = -(-(320000 // _CHUNK // _NC) // _NS)

_DOT = dict(preferred_element_type=jnp.float32,
            precision=jax.lax.Precision.DEFAULT)


_NBUF = 3


def _seg_sum_body(msg_hbm, src_hbm, dst_hbm, out_hbm, sidx, didx,
                  rows0, rows1, rows2, acc,
                  sg0, sg1, sg2, si0, si1, si2):
    c = lax.axis_index("c")
    s = lax.axis_index("s")
    E = src_hbm.shape[0]
    n_chunks = E // _CHUNK
    per_core = n_chunks // _NC
    kmax = (per_core + _NS - 1) // _NS
    slab = (N // 8 // _NS) * 8
    chunks = []
    o = 0
    while o < slab:
        sz = min(_CHUNK, slab - o)
        chunks.append((o, sz))
        o += sz

    @pl.loop(0, _CHUNK)
    def _(r):
        @pl.loop(0, H // 16)
        def _(j):
            rows0[r, pl.ds(j * 16, 16)] = jnp.zeros((16,), jnp.float32)

    row0 = pl.multiple_of(s * slab, 8)
    for o, sz in chunks:
        pltpu.sync_copy(rows0.at[pl.ds(0, sz)],
                        acc.at[pl.ds(row0 + o, sz)])

    @pl.when(s == 0)
    def _():
        r0 = pl.multiple_of(slab * _NS, 8)
        pltpu.sync_copy(rows0.at[pl.ds(0, N - slab * _NS)],
                        acc.at[pl.ds(r0, N - slab * _NS)])

    base_k = per_core // _NS
    extra = per_core - base_k * _NS
    cs = c * per_core + s * base_k + jnp.minimum(s, extra)
    kn = base_k + jnp.where(s < extra, 1, 0)
    edge0 = cs * _CHUNK

    plsc.subcore_barrier()

    bufs = ((rows0, sg0, si0), (rows1, sg1, si1), (rows2, sg2, si2))

    def _idx_copies(k, b, si_b):
        return (pltpu.make_async_copy(
                    src_hbm.at[pl.ds(edge0 + k * _CHUNK, _CHUNK)],
                    sidx.at[b], si_b),
                pltpu.make_async_copy(
                    dst_hbm.at[pl.ds(edge0 + k * _CHUNK, _CHUNK)],
                    didx.at[b], si_b))

    def _gather_copy(k, b, rows_b, sg_b):
        return pltpu.make_async_copy(
            msg_hbm.at[sidx.at[b]], rows_b, sg_b)

    for b in range(_NBUF):
        rows_b, sg_b, si_b = bufs[b]

        @pl.when(b < kn)
        def _(b=b, si_b=si_b):
            for cp in _idx_copies(b, b, si_b):
                cp.start()

    for b in range(2):
        rows_b, sg_b, si_b = bufs[b]

        @pl.when(b < kn)
        def _(b=b, rows_b=rows_b, sg_b=sg_b, si_b=si_b):
            for cp in _idx_copies(b, b, si_b):
                cp.wait()
            _gather_copy(b, b, rows_b, sg_b).start()

    @pl.loop(0, (kmax + _NBUF - 1) // _NBUF)
    def _(kq):
        for b in range(_NBUF):
            rows_b, sg_b, si_b = bufs[b]
            b2 = (b + 2) % _NBUF
            rows_b2, sg_b2, si_b2 = bufs[b2]
            k = kq * _NBUF + b

            @pl.when(k < kn)
            def _(k=k, b=b, rows_b=rows_b, sg_b=sg_b, si_b=si_b,
                  b2=b2, rows_b2=rows_b2, sg_b2=sg_b2, si_b2=si_b2):
                _gather_copy(k, b, rows_b, sg_b).wait()
                pltpu.sync_copy(rows_b, acc.at[didx.at[b]], add=True)

                @pl.when(k + _NBUF < kn)
                def _():
                    for cp in _idx_copies(k + _NBUF, b, si_b):
                        cp.start()

                @pl.when(k + 2 < kn)
                def _():
                    for cp in _idx_copies(k + 2, b2, si_b2):
                        cp.wait()
                    _gather_copy(k + 2, b2, rows_b2, sg_b2).start()

    plsc.subcore_barrier()

    for o, sz in chunks:
        r1 = pl.multiple_of(row0 + o, 8)
        pltpu.sync_copy(acc.at[pl.ds(r1, sz)],
                        out_hbm.at[pl.ds(pl.multiple_of(c * N + r1, 8), sz)])

    @pl.when(s == 0)
    def _():
        r0 = pl.multiple_of(slab * _NS, 8)
        pltpu.sync_copy(
            acc.at[pl.ds(r0, N - slab * _NS)],
            out_hbm.at[pl.ds(pl.multiple_of(c * N + r0, 8), N - slab * _NS)])


def _seg_sum(msg, src, dst):
    mesh = plsc.VectorSubcoreMesh(core_axis_name="c", subcore_axis_name="s")
    f = pl.kernel(
        _seg_sum_body,
        out_type=jax.ShapeDtypeStruct((_NC * N, H), jnp.float32),
        mesh=mesh,
        scratch_types=[
            pltpu.VMEM((_NBUF, _CHUNK), jnp.int32),
            pltpu.VMEM((_NBUF, _CHUNK), jnp.int32),
            pltpu.VMEM((_CHUNK, H), jnp.float32),
            pltpu.VMEM((_CHUNK, H), jnp.float32),
            pltpu.VMEM((_CHUNK, H), jnp.float32),
            pltpu.VMEM_SHARED((N, H), jnp.float32),
            pltpu.SemaphoreType.DMA,
            pltpu.SemaphoreType.DMA,
            pltpu.SemaphoreType.DMA,
            pltpu.SemaphoreType.DMA,
            pltpu.SemaphoreType.DMA,
            pltpu.SemaphoreType.DMA,
        ],
    )
    return f(msg, src, dst)


_RB = 2000


def _gru_body(p0_ref, p1_ref, h_ref, w_ref, wih_ref, whh_ref, bih_ref,
              bhh_ref, o_ref):
    agg = p0_ref[...] + p1_ref[...]
    m = lax.dot_general(agg, w_ref[...], (((1,), (0,)), ((), ())), **_DOT)
    h = h_ref[...]
    gi = lax.dot_general(m, wih_ref[...], (((1,), (1,)), ((), ())), **_DOT)
    gi = gi + bih_ref[...][None, :]
    gh = lax.dot_general(h, whh_ref[...], (((1,), (1,)), ((), ())), **_DOT)
    gh = gh + bhh_ref[...][None, :]
    r = jax.nn.sigmoid(gi[:, 0:H] + gh[:, 0:H])
    z = jax.nn.sigmoid(gi[:, H:2 * H] + gh[:, H:2 * H])
    n = jnp.tanh(gi[:, 2 * H:3 * H] + r * gh[:, 2 * H:3 * H])
    o_ref[...] = (1.0 - z) * n + z * h


def _gru_call(parts, h, w, w_ih, w_hh, b_ih, b_hh):
    nb = N // _RB
    return pl.pallas_call(
        _gru_body,
        grid=(nb,),
        in_specs=[pl.BlockSpec((_RB, H), lambda i: (i, 0)),
                  pl.BlockSpec((_RB, H), lambda i, nb=nb: (i + nb, 0)),
                  pl.BlockSpec((_RB, H), lambda i: (i, 0)),
                  pl.BlockSpec((H, H), lambda i: (0, 0)),
                  pl.BlockSpec((3 * H, H), lambda i: (0, 0)),
                  pl.BlockSpec((3 * H, H), lambda i: (0, 0)),
                  pl.BlockSpec((3 * H,), lambda i: (0,)),
                  pl.BlockSpec((3 * H,), lambda i: (0,))],
        out_specs=pl.BlockSpec((_RB, H), lambda i: (i, 0)),
        out_shape=jax.ShapeDtypeStruct((N, H), jnp.float32),
    )(parts, parts, h, w, w_ih, w_hh, b_ih, b_hh)


def _gru_tail_body(p0_ref, p1_ref, h_ref, w_ref, wih_ref, whh_ref, bih_ref,
                   bhh_ref, w1_ref, b1_ref, bng_ref, bnb_ref, batch_ref,
                   w2_ref, b2_ref, o_ref, y_scr, st_scr):
    nb = N // _RB
    i = pl.program_id(0)

    @pl.when(i < nb)
    def _():
        agg = p0_ref[...] + p1_ref[...]
        m = lax.dot_general(agg, w_ref[...], (((1,), (0,)), ((), ())), **_DOT)
        h = h_ref[...]
        gi = lax.dot_general(m, wih_ref[...], (((1,), (1,)), ((), ())),
                             **_DOT) + bih_ref[...][None, :]
        gh = lax.dot_general(h, whh_ref[...], (((1,), (1,)), ((), ())),
                             **_DOT) + bhh_ref[...][None, :]
        r = jax.nn.sigmoid(gi[:, 0:H] + gh[:, 0:H])
        z = jax.nn.sigmoid(gi[:, H:2 * H] + gh[:, H:2 * H])
        n = jnp.tanh(gi[:, 2 * H:3 * H] + r * gh[:, 2 * H:3 * H])
        hn = (1.0 - z) * n + z * h
        y = lax.dot_general(hn, w1_ref[...], (((1,), (1,)), ((), ())),
                            **_DOT) + b1_ref[...][None, :]
        y_scr[pl.ds(pl.multiple_of(i * _RB, 8), _RB), :] = y
        st = jnp.stack([jnp.sum(y, axis=0), jnp.sum(y * y, axis=0)], axis=0)

        @pl.when(i == 0)
        def _():
            st_scr[...] = st

        @pl.when(i != 0)
        def _():
            st_scr[...] += st

    @pl.when(i == nb)
    def _():
        mean = st_scr[0, :] / N
        var = st_scr[1, :] / N - mean * mean
        scale = bng_ref[...] * lax.rsqrt(var + 1e-5)
        y = (y_scr[...] - mean[None, :]) * scale[None, :]
        y = jnp.maximum(y + bnb_ref[...][None, :], 0.0)
        b = batch_ref[0, :]
        onehot = (b[:, None] == lax.broadcasted_iota(jnp.int32, (N, G), 1))
        onehot = onehot.astype(jnp.float32)
        gs = lax.dot_general(onehot, y, (((0,), (0,)), ((), ())), **_DOT)
        gc = jnp.sum(onehot, axis=0)
        gm = gs / jnp.maximum(gc, 1.0)[:, None]
        logits = lax.dot_general(gm, w2_ref[...], (((1,), (1,)), ((), ())),
                                 **_DOT) + b2_ref[...][None, :]
        mx = jnp.max(logits, axis=-1, keepdims=True)
        sh = logits - mx
        lse = jnp.log(jnp.sum(jnp.exp(sh), axis=-1, keepdims=True))
        o_ref[...] = sh - lse


def _gru_tail_call(parts, h, w, w_ih, w_hh, b_ih, b_hh,
                   fc1_w, fc1_b, bn_g, bn_b, batch, fc2_w, fc2_b):
    C = fc2_w.shape[0]
    nb = N // _RB
    blk = lambda i: (jnp.minimum(i, nb - 1), 0)
    blk2 = lambda i: (jnp.minimum(i, nb - 1) + nb, 0)
    full = lambda i: (0, 0)
    vec = lambda i: (0,)
    return pl.pallas_call(
        _gru_tail_body,
        grid=(nb + 1,),
        in_specs=[pl.BlockSpec((_RB, H), blk),
                  pl.BlockSpec((_RB, H), blk2),
                  pl.BlockSpec((_RB, H), blk),
                  pl.BlockSpec((H, H), full),
                  pl.BlockSpec((3 * H, H), full),
                  pl.BlockSpec((3 * H, H), full),
                  pl.BlockSpec((3 * H,), vec),
                  pl.BlockSpec((3 * H,), vec),
                  pl.BlockSpec((H, H), full),
                  pl.BlockSpec((H,), vec),
                  pl.BlockSpec((H,), vec),
                  pl.BlockSpec((H,), vec),
                  pl.BlockSpec((1, N), full),
                  pl.BlockSpec((C, H), full),
                  pl.BlockSpec((C,), vec)],
        out_specs=pl.BlockSpec((G, C), full),
        out_shape=jax.ShapeDtypeStruct((G, C), jnp.float32),
        scratch_shapes=[pltpu.VMEM((N, H), jnp.float32),
                        pltpu.VMEM((2, H), jnp.float32)],
    )(parts, parts, h, w, w_ih, w_hh, b_ih, b_hh,
      fc1_w, fc1_b, bn_g, bn_b, batch.reshape(1, N), fc2_w, fc2_b)


def kernel(x, edge_index, batch, weight, w_ih, w_hh, b_ih, b_hh,
           fc1_w, fc1_b, bn_g, bn_b, fc2_w, fc2_b):
    src = edge_index[0]
    dst = edge_index[1]
    h = x
    L = weight.shape[0]
    for l in range(L - 1):
        parts = _seg_sum(h, src, dst)
        h = _gru_call(parts, h, weight[l], w_ih, w_hh, b_ih, b_hh)
    parts = _seg_sum(h, src, dst)
    return _gru_tail_call(parts, h, weight[L - 1], w_ih, w_hh, b_ih, b_hh,
                          fc1_w, fc1_b, bn_g, bn_b, batch, fc2_w, fc2_b)


# speedup vs baseline: 12.4545x; 1.0015x over previous
import functools

import jax
import jax.numpy as jnp
from jax import lax
from jax.experimental import pallas as pl
from jax.experimental.pallas import tpu as pltpu
from jax.experimental.pallas import tpu_sc as plsc

N = 10000
H = 128
G = 64

_NC = 2
_NS = 16
_CHUNK = 128
_KMAX = -(-(320000 // _CHUNK // _NC) // _NS)

_DOT = dict(preferred_element_type=jnp.float32,
            precision=jax.lax.Precision.DEFAULT)


_NBUF = 3


def _seg_sum_body(msg_hbm, src_hbm, dst_hbm, out_hbm, sidx, didx,
                  rows0, rows1, rows2, acc,
                  sg0, sg1, sg2, si0, si1, si2):
    c = lax.axis_index("c")
    s = lax.axis_index("s")
    E = src_hbm.shape[0]
    n_chunks = E // _CHUNK
    per_core = n_chunks // _NC
    kmax = (per_core + _NS - 1) // _NS
    slab = (N // 8 // _NS) * 8
    chunks = []
    o = 0
    while o < slab:
        sz = min(_CHUNK, slab - o)
        chunks.append((o, sz))
        o += sz

    @pl.loop(0, _CHUNK)
    def _(r):
        @pl.loop(0, H // 16)
        def _(j):
            rows0[r, pl.ds(j * 16, 16)] = jnp.zeros((16,), jnp.float32)

    row0 = pl.multiple_of(s * slab, 8)
    for o, sz in chunks:
        pltpu.sync_copy(rows0.at[pl.ds(0, sz)],
                        acc.at[pl.ds(row0 + o, sz)])

    @pl.when(s == 0)
    def _():
        r0 = pl.multiple_of(slab * _NS, 8)
        pltpu.sync_copy(rows0.at[pl.ds(0, N - slab * _NS)],
                        acc.at[pl.ds(r0, N - slab * _NS)])

    base_k = per_core // _NS
    extra = per_core - base_k * _NS
    cs = c * per_core + s * base_k + jnp.minimum(s, extra)
    kn = base_k + jnp.where(s < extra, 1, 0)
    edge0 = cs * _CHUNK

    bufs = ((rows0, sg0, si0), (rows1, sg1, si1), (rows2, sg2, si2))

    def _idx_copies(k, b, si_b):
        return (pltpu.make_async_copy(
                    src_hbm.at[pl.ds(edge0 + k * _CHUNK, _CHUNK)],
                    sidx.at[b], si_b),
                pltpu.make_async_copy(
                    dst_hbm.at[pl.ds(edge0 + k * _CHUNK, _CHUNK)],
                    didx.at[b], si_b))

    def _gather_copy(k, b, rows_b, sg_b):
        return pltpu.make_async_copy(
            msg_hbm.at[sidx.at[b]], rows_b, sg_b)

    for b in range(_NBUF):
        rows_b, sg_b, si_b = bufs[b]

        @pl.when(b < kn)
        def _(b=b, si_b=si_b):
            for cp in _idx_copies(b, b, si_b):
                cp.start()

    for b in range(2):
        rows_b, sg_b, si_b = bufs[b]

        @pl.when(b < kn)
        def _(b=b, rows_b=rows_b, sg_b=sg_b, si_b=si_b):
            for cp in _idx_copies(b, b, si_b):
                cp.wait()
            _gather_copy(b, b, rows_b, sg_b).start()

    plsc.subcore_barrier()

    @pl.loop(0, (kmax + _NBUF - 1) // _NBUF)
    def _(kq):
        for b in range(_NBUF):
            rows_b, sg_b, si_b = bufs[b]
            b2 = (b + 2) % _NBUF
            rows_b2, sg_b2, si_b2 = bufs[b2]
            k = kq * _NBUF + b

            @pl.when(k < kn)
            def _(k=k, b=b, rows_b=rows_b, sg_b=sg_b, si_b=si_b,
                  b2=b2, rows_b2=rows_b2, sg_b2=sg_b2, si_b2=si_b2):
                _gather_copy(k, b, rows_b, sg_b).wait()
                pltpu.sync_copy(rows_b, acc.at[didx.at[b]], add=True)

                @pl.when(k + _NBUF < kn)
                def _():
                    for cp in _idx_copies(k + _NBUF, b, si_b):
                        cp.start()

                @pl.when(k + 2 < kn)
                def _():
                    for cp in _idx_copies(k + 2, b2, si_b2):
                        cp.wait()
                    _gather_copy(k + 2, b2, rows_b2, sg_b2).start()

    plsc.subcore_barrier()

    for o, sz in chunks:
        r1 = pl.multiple_of(row0 + o, 8)
        pltpu.sync_copy(acc.at[pl.ds(r1, sz)],
                        out_hbm.at[pl.ds(pl.multiple_of(c * N + r1, 8), sz)])

    @pl.when(s == 0)
    def _():
        r0 = pl.multiple_of(slab * _NS, 8)
        pltpu.sync_copy(
            acc.at[pl.ds(r0, N - slab * _NS)],
            out_hbm.at[pl.ds(pl.multiple_of(c * N + r0, 8), N - slab * _NS)])


def _seg_sum(msg, src, dst):
    mesh = plsc.VectorSubcoreMesh(core_axis_name="c", subcore_axis_name="s")
    f = pl.kernel(
        _seg_sum_body,
        out_type=jax.ShapeDtypeStruct((_NC * N, H), jnp.float32),
        mesh=mesh,
        scratch_types=[
            pltpu.VMEM((_NBUF, _CHUNK), jnp.int32),
            pltpu.VMEM((_NBUF, _CHUNK), jnp.int32),
            pltpu.VMEM((_CHUNK, H), jnp.float32),
            pltpu.VMEM((_CHUNK, H), jnp.float32),
            pltpu.VMEM((_CHUNK, H), jnp.float32),
            pltpu.VMEM_SHARED((N, H), jnp.float32),
            pltpu.SemaphoreType.DMA,
            pltpu.SemaphoreType.DMA,
            pltpu.SemaphoreType.DMA,
            pltpu.SemaphoreType.DMA,
            pltpu.SemaphoreType.DMA,
            pltpu.SemaphoreType.DMA,
        ],
    )
    return f(msg, src, dst)


_RB = 2000


def _gru_body(p0_ref, p1_ref, h_ref, w_ref, wih_ref, whh_ref, bih_ref,
              bhh_ref, o_ref):
    agg = p0_ref[...] + p1_ref[...]
    m = lax.dot_general(agg, w_ref[...], (((1,), (0,)), ((), ())), **_DOT)
    h = h_ref[...]
    gi = lax.dot_general(m, wih_ref[...], (((1,), (1,)), ((), ())), **_DOT)
    gi = gi + bih_ref[...][None, :]
    gh = lax.dot_general(h, whh_ref[...], (((1,), (1,)), ((), ())), **_DOT)
    gh = gh + bhh_ref[...][None, :]
    r = jax.nn.sigmoid(gi[:, 0:H] + gh[:, 0:H])
    z = jax.nn.sigmoid(gi[:, H:2 * H] + gh[:, H:2 * H])
    n = jnp.tanh(gi[:, 2 * H:3 * H] + r * gh[:, 2 * H:3 * H])
    o_ref[...] = (1.0 - z) * n + z * h


def _gru_call(parts, h, w, w_ih, w_hh, b_ih, b_hh):
    nb = N // _RB
    return pl.pallas_call(
        _gru_body,
        grid=(nb,),
        in_specs=[pl.BlockSpec((_RB, H), lambda i: (i, 0)),
                  pl.BlockSpec((_RB, H), lambda i, nb=nb: (i + nb, 0)),
                  pl.BlockSpec((_RB, H), lambda i: (i, 0)),
                  pl.BlockSpec((H, H), lambda i: (0, 0)),
                  pl.BlockSpec((3 * H, H), lambda i: (0, 0)),
                  pl.BlockSpec((3 * H, H), lambda i: (0, 0)),
                  pl.BlockSpec((3 * H,), lambda i: (0,)),
                  pl.BlockSpec((3 * H,), lambda i: (0,))],
        out_specs=pl.BlockSpec((_RB, H), lambda i: (i, 0)),
        out_shape=jax.ShapeDtypeStruct((N, H), jnp.float32),
    )(parts, parts, h, w, w_ih, w_hh, b_ih, b_hh)


def _gru_tail_body(p0_ref, p1_ref, h_ref, w_ref, wih_ref, whh_ref, bih_ref,
                   bhh_ref, w1_ref, b1_ref, bng_ref, bnb_ref, batch_ref,
                   w2_ref, b2_ref, o_ref, y_scr, st_scr):
    nb = N // _RB
    i = pl.program_id(0)

    @pl.when(i < nb)
    def _():
        agg = p0_ref[...] + p1_ref[...]
        m = lax.dot_general(agg, w_ref[...], (((1,), (0,)), ((), ())), **_DOT)
        h = h_ref[...]
        gi = lax.dot_general(m, wih_ref[...], (((1,), (1,)), ((), ())),
                             **_DOT) + bih_ref[...][None, :]
        gh = lax.dot_general(h, whh_ref[...], (((1,), (1,)), ((), ())),
                             **_DOT) + bhh_ref[...][None, :]
        r = jax.nn.sigmoid(gi[:, 0:H] + gh[:, 0:H])
        z = jax.nn.sigmoid(gi[:, H:2 * H] + gh[:, H:2 * H])
        n = jnp.tanh(gi[:, 2 * H:3 * H] + r * gh[:, 2 * H:3 * H])
        hn = (1.0 - z) * n + z * h
        y = lax.dot_general(hn, w1_ref[...], (((1,), (1,)), ((), ())),
                            **_DOT) + b1_ref[...][None, :]
        y_scr[pl.ds(pl.multiple_of(i * _RB, 8), _RB), :] = y
        st = jnp.stack([jnp.sum(y, axis=0), jnp.sum(y * y, axis=0)], axis=0)

        @pl.when(i == 0)
        def _():
            st_scr[...] = st

        @pl.when(i != 0)
        def _():
            st_scr[...] += st

    @pl.when(i == nb)
    def _():
        mean = st_scr[0, :] / N
        var = st_scr[1, :] / N - mean * mean
        scale = bng_ref[...] * lax.rsqrt(var + 1e-5)
        y = (y_scr[...] - mean[None, :]) * scale[None, :]
        y = jnp.maximum(y + bnb_ref[...][None, :], 0.0)
        b = batch_ref[0, :]
        onehot = (b[:, None] == lax.broadcasted_iota(jnp.int32, (N, G), 1))
        onehot = onehot.astype(jnp.float32)
        gs = lax.dot_general(onehot, y, (((0,), (0,)), ((), ())), **_DOT)
        gc = jnp.sum(onehot, axis=0)
        gm = gs / jnp.maximum(gc, 1.0)[:, None]
        logits = lax.dot_general(gm, w2_ref[...], (((1,), (1,)), ((), ())),
                                 **_DOT) + b2_ref[...][None, :]
        mx = jnp.max(logits, axis=-1, keepdims=True)
        sh = logits - mx
        lse = jnp.log(jnp.sum(jnp.exp(sh), axis=-1, keepdims=True))
        o_ref[...] = sh - lse


def _gru_tail_call(parts, h, w, w_ih, w_hh, b_ih, b_hh,
                   fc1_w, fc1_b, bn_g, bn_b, batch, fc2_w, fc2_b):
    C = fc2_w.shape[0]
    nb = N // _RB
    blk = lambda i: (jnp.minimum(i, nb - 1), 0)
    blk2 = lambda i: (jnp.minimum(i, nb - 1) + nb, 0)
    full = lambda i: (0, 0)
    vec = lambda i: (0,)
    return pl.pallas_call(
        _gru_tail_body,
        grid=(nb + 1,),
        in_specs=[pl.BlockSpec((_RB, H), blk),
                  pl.BlockSpec((_RB, H), blk2),
                  pl.BlockSpec((_RB, H), blk),
                  pl.BlockSpec((H, H), full),
                  pl.BlockSpec((3 * H, H), full),
                  pl.BlockSpec((3 * H, H), full),
                  pl.BlockSpec((3 * H,), vec),
                  pl.BlockSpec((3 * H,), vec),
                  pl.BlockSpec((H, H), full),
                  pl.BlockSpec((H,), vec),
                  pl.BlockSpec((H,), vec),
                  pl.BlockSpec((H,), vec),
                  pl.BlockSpec((1, N), full),
                  pl.BlockSpec((C, H), full),
                  pl.BlockSpec((C,), vec)],
        out_specs=pl.BlockSpec((G, C), full),
        out_shape=jax.ShapeDtypeStruct((G, C), jnp.float32),
        scratch_shapes=[pltpu.VMEM((N, H), jnp.float32),
                        pltpu.VMEM((2, H), jnp.float32)],
    )(parts, parts, h, w, w_ih, w_hh, b_ih, b_hh,
      fc1_w, fc1_b, bn_g, bn_b, batch.reshape(1, N), fc2_w, fc2_b)


def kernel(x, edge_index, batch, weight, w_ih, w_hh, b_ih, b_hh,
           fc1_w, fc1_b, bn_g, bn_b, fc2_w, fc2_b):
    src = edge_index[0]
    dst = edge_index[1]
    h = x
    L = weight.shape[0]
    for l in range(L - 1):
        parts = _seg_sum(h, src, dst)
        h = _gru_call(parts, h, weight[l], w_ih, w_hh, b_ih, b_hh)
    parts = _seg_sum(h, src, dst)
    return _gru_tail_call(parts, h, weight[L - 1], w_ih, w_hh, b_ih, b_hh,
                          fc1_w, fc1_b, bn_g, bn_b, batch, fc2_w, fc2_b)
